# baseline (speedup 1.0000x reference)
import jax
import jax.numpy as jnp
from jax.experimental import pallas as pl

_N_MID_LAYERS = 2


def _tconv(x, edge_index, edge_attr, p):
    src = edge_index[0]
    dst = edge_index[1]
    dout = p["Wq"].shape[1]
    n = x.shape[0]
    q = x @ p["Wq"] + p["bq"]
    k = x @ p["Wk"] + p["bk"]
    v = x @ p["Wv"] + p["bv"]
    e = edge_attr @ p["We"]
    kj = k[src] + e
    vj = v[src] + e
    qi = q[dst]
    alpha = jnp.sum(qi * kj, axis=-1) / jnp.sqrt(float(dout))
    m = jax.ops.segment_max(alpha, dst, num_segments=n)
    m = jnp.where(jnp.isfinite(m), m, 0.0)
    ex = jnp.exp(alpha - m[dst])
    den = jax.ops.segment_sum(ex, dst, num_segments=n)
    a = ex / (den[dst] + 1e-16)
    out = jax.ops.segment_sum(vj * a[:, None], dst, num_segments=n)
    out = out + x @ p["Ws"] + p["bs"]
    return out


def kernel(x, edge_index, edge_attr, params):
    h = jax.nn.gelu(_tconv(x, edge_index, edge_attr, params[0]), approximate=False)
    for i in range(1, 1 + _N_MID_LAYERS):
        identity = h
        h = jax.nn.gelu(_tconv(h, edge_index, edge_attr, params[i]), approximate=False)
        h = h + identity
    return _tconv(h, edge_index, edge_attr, params[-1])


# speedup vs baseline: 4.3611x; 4.3611x over previous
import functools

import jax
import jax.numpy as jnp
from jax import lax
from jax.experimental import pallas as pl
from jax.experimental.pallas import tpu as pltpu
from jax.experimental.pallas import tpu_sc as plsc

_N = 10000
_E = 320000
_DIN = 128
_DEDGE = 16
_NMID = 2

_NPT = 320
_NPAD = 32 * _NPT
_CH = 128
_SHIFT = 256
_EPAD = _E + 2 * _CH
_ALEN = _E + 32 * _SHIFT + _CH
_RB = 1280


def _make_sc_edge(D):
    nreg = D // 16
    mesh = plsc.VectorSubcoreMesh(core_axis_name="c", subcore_axis_name="s")
    f32 = jnp.float32
    out_type = (
        jax.ShapeDtypeStruct((_NPAD, D), f32),
        jax.ShapeDtypeStruct((_NPAD,), f32),
        jax.ShapeDtypeStruct((_NPAD * 16,), f32),
        jax.ShapeDtypeStruct((_ALEN,), f32),
    )
    scratch = [
        pltpu.VMEM((_NPT, D), f32),
        pltpu.VMEM((_NPT * 16,), f32),
        pltpu.VMEM((328,), jnp.int32),
        pltpu.VMEM((_NPT,), f32),
        pltpu.VMEM((_NPT,), f32),
        pltpu.VMEM((_CH,), jnp.int32),
        pltpu.VMEM((_CH,), jnp.int32),
        pltpu.VMEM((_CH, D), f32),
        pltpu.VMEM((_CH * 16,), f32),
        pltpu.VMEM((_CH,), f32),
        pltpu.VMEM((_CH,), f32),
        pltpu.SemaphoreType.DMA,
    ]

    @functools.partial(pl.kernel, mesh=mesh, out_type=out_type,
                       scratch_types=scratch,
                       compiler_params=pltpu.CompilerParams(
                           needs_layout_passes=False))
    def sc_edge(qs_h, qe_h, k_h, v_h, srcs_h, dsts_h, ea_h, offs_h,
                z_nd_h, z_n16_h, mneg_h, zden_h,
                num_h, den_h, acc_h, alpha_h,
                qn_sl, qa_sl, offs_sl, m_ar, den_ar,
                src_b, dst_b, rows_b, ea_b, al_st, ex_st, sem):
        wid = lax.axis_index("s") * 2 + lax.axis_index("c")
        lo = wid * _NPT
        shift = wid * _SHIFT
        iota16 = lax.broadcasted_iota(jnp.int32, (16,), 0)
        lane0 = iota16 == 0

        pltpu.sync_copy(offs_h.at[pl.ds(lo, 328)], offs_sl)
        pltpu.sync_copy(qs_h.at[pl.ds(lo, _NPT)], qn_sl)
        pltpu.sync_copy(qe_h.at[pl.ds(lo * 16, _NPT * 16)], qa_sl)
        pltpu.sync_copy(mneg_h, m_ar)
        pltpu.sync_copy(zden_h, den_ar)

        off0 = plsc.load_gather(offs_sl, [jnp.zeros((16,), jnp.int32)])
        elo = jnp.min(off0)
        offn = plsc.load_gather(offs_sl, [jnp.full((16,), _NPT, jnp.int32)])
        ehi = jnp.min(offn)
        abase = (elo // 8) * 8
        nch = (ehi - abase + (_CH - 1)) // _CH

        def ph1(ci, _):
            cbase = abase + ci * _CH
            ist = jnp.maximum(elo - cbase, 0)
            ien = jnp.minimum(ehi - cbase, _CH)
            pltpu.sync_copy(srcs_h.at[pl.ds(cbase, _CH)], src_b)
            pltpu.sync_copy(dsts_h.at[pl.ds(cbase, _CH)], dst_b)
            pltpu.sync_copy(ea_h.at[pl.ds(cbase * 16, _CH * 16)], ea_b)
            pltpu.async_copy(k_h.at[src_b], rows_b, sem).wait()

            def edge1(i, _):
                bi = jnp.full((16,), i, jnp.int32)
                dlv = plsc.load_gather(dst_b, [bi]) - lo
                acc = None
                for j in range(nreg):
                    colv = iota16 + (16 * j)
                    kj = plsc.load_gather(rows_b, [bi, colv])
                    qj = plsc.load_gather(qn_sl, [dlv, colv])
                    acc = kj * qj if acc is None else acc + kj * qj
                ei16 = jnp.full((16,), i * 16, jnp.int32) + iota16
                eav = plsc.load_gather(ea_b, [ei16])
                qev = plsc.load_gather(qa_sl, [dlv * 16 + iota16])
                acc = acc + eav * qev
                av = jnp.full((16,), jnp.sum(acc), f32)
                plsc.store_scatter(al_st, [bi], av, mask=lane0)
                mv = plsc.load_gather(m_ar, [dlv])
                plsc.store_scatter(m_ar, [dlv], jnp.maximum(mv, av),
                                   mask=lane0)
                return 0
            lax.fori_loop(ist, ien, edge1, 0)
            pltpu.sync_copy(al_st, alpha_h.at[pl.ds(cbase + shift, _CH)])
            return 0
        lax.fori_loop(0, nch, ph1, 0)

        pltpu.sync_copy(z_nd_h, qn_sl)
        pltpu.sync_copy(z_n16_h, qa_sl)
        def ph2(ci, _):
            cbase = abase + ci * _CH
            ist = jnp.maximum(elo - cbase, 0)
            ien = jnp.minimum(ehi - cbase, _CH)
            pltpu.sync_copy(srcs_h.at[pl.ds(cbase, _CH)], src_b)
            pltpu.sync_copy(dsts_h.at[pl.ds(cbase, _CH)], dst_b)
            pltpu.sync_copy(ea_h.at[pl.ds(cbase * 16, _CH * 16)], ea_b)
            pltpu.sync_copy(alpha_h.at[pl.ds(cbase + shift, _CH)], al_st)
            pltpu.async_copy(v_h.at[src_b], rows_b, sem).wait()

            for g in range(_CH // 16):
                d16 = dst_b[pl.ds(16 * g, 16)] - lo
                d16 = jnp.minimum(jnp.maximum(d16, 0), _NPT - 1)
                m16 = plsc.load_gather(m_ar, [d16])
                a16 = al_st[pl.ds(16 * g, 16)]
                ex_st[pl.ds(16 * g, 16)] = jnp.exp(a16 - m16)

            def edge2(i, _):
                bi = jnp.full((16,), i, jnp.int32)
                dlv = plsc.load_gather(dst_b, [bi]) - lo
                exv = plsc.load_gather(ex_st, [bi])
                plsc.addupdate_scatter(den_ar, [dlv], exv, mask=lane0)
                for j in range(nreg):
                    colv = iota16 + (16 * j)
                    vv = plsc.load_gather(rows_b, [bi, colv])
                    plsc.addupdate_scatter(qn_sl, [dlv, colv], exv * vv)
                ei16 = jnp.full((16,), i * 16, jnp.int32) + iota16
                eav = plsc.load_gather(ea_b, [ei16])
                plsc.addupdate_scatter(qa_sl, [dlv * 16 + iota16], exv * eav)
                return 0
            lax.fori_loop(ist, ien, edge2, 0)
            return 0
        lax.fori_loop(0, nch, ph2, 0)

        pltpu.sync_copy(qn_sl, num_h.at[pl.ds(lo, _NPT)])
        pltpu.sync_copy(den_ar, den_h.at[pl.ds(lo, _NPT)])
        pltpu.sync_copy(qa_sl, acc_h.at[pl.ds(lo * 16, _NPT * 16)])

    return sc_edge


_sc_edge_128 = _make_sc_edge(128)


def _proj_call(h, wq, bq, wk, bk, wv, bv, ws, bs, wet, D):
    nb = _NPAD // _RB

    def body(h_r, wq_r, bq_r, wk_r, bk_r, wv_r, bv_r, ws_r, bs_r, wet_r,
             qs_r, k_r, v_r, sk_r, qe_r):
        hb = h_r[...]
        qs = jnp.dot(hb, wq_r[...], preferred_element_type=jnp.float32) + bq_r[...]
        qs_r[...] = qs
        k_r[...] = jnp.dot(hb, wk_r[...], preferred_element_type=jnp.float32) + bk_r[...]
        v_r[...] = jnp.dot(hb, wv_r[...], preferred_element_type=jnp.float32) + bv_r[...]
        sk_r[...] = jnp.dot(hb, ws_r[...], preferred_element_type=jnp.float32) + bs_r[...]
        qe_r[...] = jnp.dot(qs, wet_r[...], preferred_element_type=jnp.float32)

    din = h.shape[1]
    wspec = pl.BlockSpec((din, D), lambda i: (0, 0))
    bspec = pl.BlockSpec((1, D), lambda i: (0, 0))
    return pl.pallas_call(
        body,
        grid=(nb,),
        in_specs=[pl.BlockSpec((_RB, din), lambda i: (i, 0)),
                  wspec, bspec, wspec, bspec, wspec, bspec, wspec, bspec,
                  pl.BlockSpec((D, 16), lambda i: (0, 0))],
        out_specs=[pl.BlockSpec((_RB, D), lambda i: (i, 0))] * 4
        + [pl.BlockSpec((_RB, 16), lambda i: (i, 0))],
        out_shape=[jax.ShapeDtypeStruct((_NPAD, D), jnp.float32)] * 4
        + [jax.ShapeDtypeStruct((_NPAD, 16), jnp.float32)],
    )(h, wq, bq, wk, bk, wv, bv, ws, bs, wet)


def _gelu_exact(x):
    return 0.5 * x * (1.0 + lax.erf(x * 0.7071067811865476))


def _combine_call(num, den2, acc, skip, we, identity, mode, D):
    nb = _NPAD // _RB

    def body(*refs):
        if mode == "mid":
            (num_r, den_r, acc_r, sk_r, we_r, id_r, o_r) = refs
        else:
            (num_r, den_r, acc_r, sk_r, we_r, o_r) = refs
        attn = num_r[...] + jnp.dot(acc_r[...], we_r[...],
                                    preferred_element_type=jnp.float32)
        o = attn / (den_r[...] + 1e-16) + sk_r[...]
        if mode == "first":
            o = _gelu_exact(o)
        elif mode == "mid":
            o = _gelu_exact(o) + id_r[...]
        o_r[...] = o

    in_specs = [pl.BlockSpec((_RB, D), lambda i: (i, 0)),
                pl.BlockSpec((_RB, 1), lambda i: (i, 0)),
                pl.BlockSpec((_RB, 16), lambda i: (i, 0)),
                pl.BlockSpec((_RB, D), lambda i: (i, 0)),
                pl.BlockSpec((16, D), lambda i: (0, 0))]
    args = [num, den2, acc, skip, we]
    if mode == "mid":
        in_specs.append(pl.BlockSpec((_RB, D), lambda i: (i, 0)))
        args.append(identity)
    return pl.pallas_call(
        body,
        grid=(nb,),
        in_specs=in_specs,
        out_specs=pl.BlockSpec((_RB, D), lambda i: (i, 0)),
        out_shape=jax.ShapeDtypeStruct((_NPAD, D), jnp.float32),
    )(*args)


def _pad_params(p):
    dout = p["Wq"].shape[1]
    D = 128
    inv = 1.0 / jnp.sqrt(jnp.float32(dout))

    def padw(w):
        if w.shape[1] == D:
            return w
        return jnp.zeros((w.shape[0], D), jnp.float32).at[:, :dout].set(w)

    def padb(b):
        if b.shape[0] == D:
            return b.reshape(1, D)
        return jnp.zeros((1, D), jnp.float32).at[0, :dout].set(b)

    wq = padw(p["Wq"]) * inv
    bq = padb(p["bq"]) * inv
    we = padw(p["We"])
    return dict(D=D, wq=wq, bq=bq, wk=padw(p["Wk"]), bk=padb(p["bk"]),
                wv=padw(p["Wv"]), bv=padb(p["bv"]), ws=padw(p["Ws"]),
                bs=padb(p["bs"]), we=we, wet=we.T)


def _layer(h, srcs, dsts, eas, offs, pp, mode, identity=None):
    D = pp["D"]
    qs, k, v, skip, qe = _proj_call(
        h, pp["wq"], pp["bq"], pp["wk"], pp["bk"], pp["wv"], pp["bv"],
        pp["ws"], pp["bs"], pp["wet"], D)
    sc = _sc_edge_128
    z_nd = jnp.zeros((_NPT, D), jnp.float32)
    z_n16 = jnp.zeros((_NPT * 16,), jnp.float32)
    mneg = jnp.full((_NPT,), -3.0e38, jnp.float32)
    zden = jnp.zeros((_NPT,), jnp.float32)
    num, den, acc, _ = sc(qs, qe.reshape(-1), k, v, srcs, dsts, eas, offs,
                          z_nd, z_n16, mneg, zden)
    return _combine_call(num, den.reshape(_NPAD, 1), acc.reshape(_NPAD, 16),
                         skip, pp["we"], identity, mode, D)


def kernel(x, edge_index, edge_attr, params):
    src = edge_index[0]
    dst = edge_index[1]
    perm = jnp.argsort(dst)
    dsts = dst[perm]
    srcs = src[perm]
    eas = edge_attr[perm]

    srcs_p = jnp.zeros((_EPAD,), jnp.int32).at[:_E].set(srcs)
    dsts_p = jnp.zeros((_EPAD,), jnp.int32).at[:_E].set(dsts)
    eas_p = jnp.zeros((_EPAD, _DEDGE), jnp.float32).at[:_E].set(eas).reshape(-1)
    offs = jnp.searchsorted(dsts, jnp.arange(_NPAD + 128, dtype=jnp.int32),
                            side="left").astype(jnp.int32)

    h = jnp.zeros((_NPAD, _DIN), jnp.float32).at[:_N].set(x)

    pps = [_pad_params(p) for p in params]
    h = _layer(h, srcs_p, dsts_p, eas_p, offs, pps[0], "first")
    for i in range(1, 1 + _NMID):
        h = _layer(h, srcs_p, dsts_p, eas_p, offs, pps[i], "mid", identity=h)
    out = _layer(h, srcs_p, dsts_p, eas_p, offs, pps[-1], "last")
    return out[:_N, :3]


# speedup vs baseline: 5.5320x; 1.2685x over previous
import functools

import jax
import jax.numpy as jnp
from jax import lax
from jax.experimental import pallas as pl
from jax.experimental.pallas import tpu as pltpu
from jax.experimental.pallas import tpu_sc as plsc

_N = 10000
_E = 320000
_DIN = 128
_DEDGE = 16
_NMID = 2

_NPT = 320
_NPAD = 32 * _NPT
_CH = 128
_SHIFT = 256
_EPAD = _E + 2 * _CH
_ALEN = _E + 32 * _SHIFT + _CH
_RB = 1280


def _make_sc_edge(D):
    nreg = D // 16
    mesh = plsc.VectorSubcoreMesh(core_axis_name="c", subcore_axis_name="s")
    f32 = jnp.float32
    out_type = (
        jax.ShapeDtypeStruct((_NPAD, D), f32),
        jax.ShapeDtypeStruct((_NPAD,), f32),
        jax.ShapeDtypeStruct((_NPAD * 16,), f32),
        jax.ShapeDtypeStruct((_ALEN,), f32),
    )
    scratch = [
        pltpu.VMEM((_NPT, D), f32),
        pltpu.VMEM((_NPT * 16,), f32),
        pltpu.VMEM((328,), jnp.int32),
        pltpu.VMEM((_NPT,), f32),
        pltpu.VMEM((_NPT,), f32),
        pltpu.VMEM((_CH,), jnp.int32),
        pltpu.VMEM((_CH,), jnp.int32),
        pltpu.VMEM((_CH,), jnp.int32),
        pltpu.VMEM((_CH,), jnp.int32),
        pltpu.VMEM((_CH, D), f32),
        pltpu.VMEM((_CH, D), f32),
        pltpu.VMEM((_CH * 16,), f32),
        pltpu.VMEM((_CH * 16,), f32),
        pltpu.VMEM((_CH,), f32),
        pltpu.VMEM((_CH,), f32),
        pltpu.VMEM((_CH,), f32),
        pltpu.SemaphoreType.DMA,
        pltpu.SemaphoreType.DMA,
        pltpu.SemaphoreType.DMA,
        pltpu.SemaphoreType.DMA,
        pltpu.SemaphoreType.DMA,
    ]

    @functools.partial(pl.kernel, mesh=mesh, out_type=out_type,
                       scratch_types=scratch,
                       compiler_params=pltpu.CompilerParams(
                           needs_layout_passes=False))
    def sc_edge(qs_h, qe_h, k_h, v_h, srcs_h, dsts_h, ea_h, offs_h,
                z_nd_h, z_n16_h, mneg_h, zden_h,
                num_h, den_h, acc_h, alpha_h,
                qn_sl, qa_sl, offs_sl, m_ar, den_ar,
                src_b0, src_b1, dst_b0, dst_b1, rows_b0, rows_b1,
                ea_b0, ea_b1, al_st0, al_st1, ex_st,
                lsem0, lsem1, rsem0, rsem1, asem):
        wid = lax.axis_index("s") * 2 + lax.axis_index("c")
        lo = wid * _NPT
        shift = wid * _SHIFT
        iota16 = lax.broadcasted_iota(jnp.int32, (16,), 0)
        lane0 = iota16 == 0
        lsem = (lsem0, lsem1)
        rsem = (rsem0, rsem1)
        src_b = (src_b0, src_b1)
        dst_b = (dst_b0, dst_b1)
        rows_b = (rows_b0, rows_b1)
        ea_b = (ea_b0, ea_b1)
        al_st = (al_st0, al_st1)

        pltpu.sync_copy(offs_h.at[pl.ds(lo, 328)], offs_sl)
        pltpu.sync_copy(qs_h.at[pl.ds(lo, _NPT)], qn_sl)
        pltpu.sync_copy(qe_h.at[pl.ds(lo * 16, _NPT * 16)], qa_sl)
        pltpu.sync_copy(mneg_h, m_ar)
        pltpu.sync_copy(zden_h, den_ar)

        off0 = plsc.load_gather(offs_sl, [jnp.zeros((16,), jnp.int32)])
        elo = jnp.min(off0)
        offn = plsc.load_gather(offs_sl, [jnp.full((16,), _NPT, jnp.int32)])
        ehi = jnp.min(offn)
        abase = (elo // 8) * 8
        nch = (ehi - abase + (_CH - 1)) // _CH

        def bounds(c):
            cb = abase + c * _CH
            ist = jnp.maximum(elo - cb, 0)
            ien = jnp.minimum(ehi - cb, _CH)
            return cb, ist, ien

        def lin_issue(c, s, with_alpha):
            cb = abase + c * _CH
            pltpu.async_copy(srcs_h.at[pl.ds(cb, _CH)], src_b[s], lsem[s])
            pltpu.async_copy(dsts_h.at[pl.ds(cb, _CH)], dst_b[s], lsem[s])
            pltpu.async_copy(ea_h.at[pl.ds(cb * 16, _CH * 16)], ea_b[s],
                             lsem[s])
            if with_alpha:
                pltpu.async_copy(alpha_h.at[pl.ds(cb + shift, _CH)],
                                 al_st[s], lsem[s])

        def lin_wait(s, with_alpha):
            pltpu.make_async_copy(srcs_h.at[pl.ds(0, _CH)], src_b[s],
                                  lsem[s]).wait()
            pltpu.make_async_copy(dsts_h.at[pl.ds(0, _CH)], dst_b[s],
                                  lsem[s]).wait()
            pltpu.make_async_copy(ea_h.at[pl.ds(0, _CH * 16)], ea_b[s],
                                  lsem[s]).wait()
            if with_alpha:
                pltpu.make_async_copy(alpha_h.at[pl.ds(0, _CH)], al_st[s],
                                      lsem[s]).wait()

        def rows_issue(s, table):
            pltpu.async_copy(table.at[src_b[s]], rows_b[s], rsem[s])

        def rows_wait(s, table):
            pltpu.make_async_copy(table.at[src_b[s]], rows_b[s],
                                  rsem[s]).wait()

        def run_phase(table, with_alpha, compute):
            @pl.when(nch > 0)
            def _():
                lin_issue(0, 0, with_alpha)

            @pl.when(nch > 1)
            def _():
                lin_issue(1, 1, with_alpha)

            @pl.when(nch > 0)
            def _():
                lin_wait(0, with_alpha)
                rows_issue(0, table)

            def pair(ci2, _):
                c0 = ci2 * 2
                c1 = c0 + 1

                @pl.when(c1 < nch)
                def _():
                    lin_wait(1, with_alpha)
                    rows_issue(1, table)
                rows_wait(0, table)
                compute(c0, 0)

                @pl.when(c0 + 2 < nch)
                def _():
                    lin_issue(c0 + 2, 0, with_alpha)

                @pl.when(c1 < nch)
                def _():
                    @pl.when(c0 + 2 < nch)
                    def _():
                        lin_wait(0, with_alpha)
                        rows_issue(0, table)
                    rows_wait(1, table)
                    compute(c1, 1)

                    @pl.when(c0 + 3 < nch)
                    def _():
                        lin_issue(c0 + 3, 1, with_alpha)
                return 0
            lax.fori_loop(0, (nch + 1) // 2, pair, 0)

        def compute1(c, s):
            cb, ist, ien = bounds(c)
            dsts_r = dst_b[s]
            ea_r = ea_b[s]
            rows_r = rows_b[s]
            al_r = al_st[s]

            @pl.when(c >= 2)
            def _():
                pltpu.make_async_copy(al_st[0],
                                      alpha_h.at[pl.ds(0, _CH)], asem).wait()

            def edge1(i, _):
                bi = jnp.full((16,), i, jnp.int32)
                dlv = plsc.load_gather(dsts_r, [bi]) - lo
                acc = None
                for j in range(nreg):
                    colv = iota16 + (16 * j)
                    kj = plsc.load_gather(rows_r, [bi, colv])
                    qj = plsc.load_gather(qn_sl, [dlv, colv])
                    acc = kj * qj if acc is None else acc + kj * qj
                ei16 = jnp.full((16,), i * 16, jnp.int32) + iota16
                eav = plsc.load_gather(ea_r, [ei16])
                qev = plsc.load_gather(qa_sl, [dlv * 16 + iota16])
                acc = acc + eav * qev
                av = jnp.full((16,), jnp.sum(acc), f32)
                plsc.store_scatter(al_r, [bi], av, mask=lane0)
                return 0
            lax.fori_loop(ist, ien, edge1, 0)

            for g in range(_CH // 16):
                lanev = iota16 + (16 * g)
                valid = (lanev >= ist) & (lanev < ien)
                d16 = dsts_r[pl.ds(16 * g, 16)] - lo
                d16 = jnp.minimum(jnp.maximum(d16, 0), _NPT - 1)
                a16 = al_r[pl.ds(16 * g, 16)]

                def mbody(rem):
                    cur = plsc.load_gather(m_ar, [d16])
                    new = jnp.maximum(cur, a16)
                    plsc.store_scatter(m_ar, [d16], new, mask=rem)
                    chk = plsc.load_gather(m_ar, [d16])
                    return rem & (chk < new)
                lax.while_loop(jnp.any, mbody, valid)

            pltpu.async_copy(al_r, alpha_h.at[pl.ds(cb + shift, _CH)], asem)

        run_phase(k_h, False, compute1)

        @pl.when(nch >= 2)
        def _():
            pltpu.make_async_copy(al_st[0], alpha_h.at[pl.ds(0, _CH)],
                                  asem).wait()

        @pl.when(nch >= 1)
        def _():
            pltpu.make_async_copy(al_st[0], alpha_h.at[pl.ds(0, _CH)],
                                  asem).wait()

        pltpu.sync_copy(z_nd_h, qn_sl)
        pltpu.sync_copy(z_n16_h, qa_sl)

        def compute2(c, s):
            cb, ist, ien = bounds(c)
            dsts_r = dst_b[s]
            ea_r = ea_b[s]
            rows_r = rows_b[s]
            al_r = al_st[s]

            for g in range(_CH // 16):
                d16 = dsts_r[pl.ds(16 * g, 16)] - lo
                d16 = jnp.minimum(jnp.maximum(d16, 0), _NPT - 1)
                m16 = plsc.load_gather(m_ar, [d16])
                a16 = al_r[pl.ds(16 * g, 16)]
                ex_st[pl.ds(16 * g, 16)] = jnp.exp(a16 - m16)

            def edge2(i, _):
                bi = jnp.full((16,), i, jnp.int32)
                dlv = plsc.load_gather(dsts_r, [bi]) - lo
                exv = plsc.load_gather(ex_st, [bi])
                plsc.addupdate_scatter(den_ar, [dlv], exv, mask=lane0)
                for j in range(nreg):
                    colv = iota16 + (16 * j)
                    vv = plsc.load_gather(rows_r, [bi, colv])
                    plsc.addupdate_scatter(qn_sl, [dlv, colv], exv * vv)
                ei16 = jnp.full((16,), i * 16, jnp.int32) + iota16
                eav = plsc.load_gather(ea_r, [ei16])
                plsc.addupdate_scatter(qa_sl, [dlv * 16 + iota16], exv * eav)
                return 0
            lax.fori_loop(ist, ien, edge2, 0)

        run_phase(v_h, True, compute2)

        pltpu.sync_copy(qn_sl, num_h.at[pl.ds(lo, _NPT)])
        pltpu.sync_copy(den_ar, den_h.at[pl.ds(lo, _NPT)])
        pltpu.sync_copy(qa_sl, acc_h.at[pl.ds(lo * 16, _NPT * 16)])

    return sc_edge


_sc_edge_128 = _make_sc_edge(128)


def _proj_call(h, wq, bq, wk, bk, wv, bv, ws, bs, wet, D):
    nb = _NPAD // _RB

    def body(h_r, wq_r, bq_r, wk_r, bk_r, wv_r, bv_r, ws_r, bs_r, wet_r,
             qs_r, k_r, v_r, sk_r, qe_r):
        hb = h_r[...]
        qs = jnp.dot(hb, wq_r[...], preferred_element_type=jnp.float32) + bq_r[...]
        qs_r[...] = qs
        k_r[...] = jnp.dot(hb, wk_r[...], preferred_element_type=jnp.float32) + bk_r[...]
        v_r[...] = jnp.dot(hb, wv_r[...], preferred_element_type=jnp.float32) + bv_r[...]
        sk_r[...] = jnp.dot(hb, ws_r[...], preferred_element_type=jnp.float32) + bs_r[...]
        qe_r[...] = jnp.dot(qs, wet_r[...], preferred_element_type=jnp.float32)

    din = h.shape[1]
    wspec = pl.BlockSpec((din, D), lambda i: (0, 0))
    bspec = pl.BlockSpec((1, D), lambda i: (0, 0))
    return pl.pallas_call(
        body,
        grid=(nb,),
        in_specs=[pl.BlockSpec((_RB, din), lambda i: (i, 0)),
                  wspec, bspec, wspec, bspec, wspec, bspec, wspec, bspec,
                  pl.BlockSpec((D, 16), lambda i: (0, 0))],
        out_specs=[pl.BlockSpec((_RB, D), lambda i: (i, 0))] * 4
        + [pl.BlockSpec((_RB, 16), lambda i: (i, 0))],
        out_shape=[jax.ShapeDtypeStruct((_NPAD, D), jnp.float32)] * 4
        + [jax.ShapeDtypeStruct((_NPAD, 16), jnp.float32)],
    )(h, wq, bq, wk, bk, wv, bv, ws, bs, wet)


def _gelu_exact(x):
    return 0.5 * x * (1.0 + lax.erf(x * 0.7071067811865476))


def _combine_call(num, den2, acc, skip, we, identity, mode, D):
    nb = _NPAD // _RB

    def body(*refs):
        if mode == "mid":
            (num_r, den_r, acc_r, sk_r, we_r, id_r, o_r) = refs
        else:
            (num_r, den_r, acc_r, sk_r, we_r, o_r) = refs
        attn = num_r[...] + jnp.dot(acc_r[...], we_r[...],
                                    preferred_element_type=jnp.float32)
        o = attn / (den_r[...] + 1e-16) + sk_r[...]
        if mode == "first":
            o = _gelu_exact(o)
        elif mode == "mid":
            o = _gelu_exact(o) + id_r[...]
        o_r[...] = o

    in_specs = [pl.BlockSpec((_RB, D), lambda i: (i, 0)),
                pl.BlockSpec((_RB, 1), lambda i: (i, 0)),
                pl.BlockSpec((_RB, 16), lambda i: (i, 0)),
                pl.BlockSpec((_RB, D), lambda i: (i, 0)),
                pl.BlockSpec((16, D), lambda i: (0, 0))]
    args = [num, den2, acc, skip, we]
    if mode == "mid":
        in_specs.append(pl.BlockSpec((_RB, D), lambda i: (i, 0)))
        args.append(identity)
    return pl.pallas_call(
        body,
        grid=(nb,),
        in_specs=in_specs,
        out_specs=pl.BlockSpec((_RB, D), lambda i: (i, 0)),
        out_shape=jax.ShapeDtypeStruct((_NPAD, D), jnp.float32),
    )(*args)


def _pad_params(p):
    dout = p["Wq"].shape[1]
    D = 128
    inv = 1.0 / jnp.sqrt(jnp.float32(dout))

    def padw(w):
        if w.shape[1] == D:
            return w
        return jnp.zeros((w.shape[0], D), jnp.float32).at[:, :dout].set(w)

    def padb(b):
        if b.shape[0] == D:
            return b.reshape(1, D)
        return jnp.zeros((1, D), jnp.float32).at[0, :dout].set(b)

    wq = padw(p["Wq"]) * inv
    bq = padb(p["bq"]) * inv
    we = padw(p["We"])
    return dict(D=D, wq=wq, bq=bq, wk=padw(p["Wk"]), bk=padb(p["bk"]),
                wv=padw(p["Wv"]), bv=padb(p["bv"]), ws=padw(p["Ws"]),
                bs=padb(p["bs"]), we=we, wet=we.T)


def _layer(h, srcs, dsts, eas, offs, pp, mode, identity=None):
    D = pp["D"]
    qs, k, v, skip, qe = _proj_call(
        h, pp["wq"], pp["bq"], pp["wk"], pp["bk"], pp["wv"], pp["bv"],
        pp["ws"], pp["bs"], pp["wet"], D)
    sc = _sc_edge_128
    z_nd = jnp.zeros((_NPT, D), jnp.float32)
    z_n16 = jnp.zeros((_NPT * 16,), jnp.float32)
    mneg = jnp.full((_NPT,), -3.0e38, jnp.float32)
    zden = jnp.zeros((_NPT,), jnp.float32)
    num, den, acc, _ = sc(qs, qe.reshape(-1), k, v, srcs, dsts, eas, offs,
                          z_nd, z_n16, mneg, zden)
    return _combine_call(num, den.reshape(_NPAD, 1), acc.reshape(_NPAD, 16),
                         skip, pp["we"], identity, mode, D)


def kernel(x, edge_index, edge_attr, params):
    src = edge_index[0]
    dst = edge_index[1]
    perm = jnp.argsort(dst)
    dsts = dst[perm]
    srcs = src[perm]
    eas = edge_attr[perm]

    srcs_p = jnp.zeros((_EPAD,), jnp.int32).at[:_E].set(srcs)
    dsts_p = jnp.zeros((_EPAD,), jnp.int32).at[:_E].set(dsts)
    eas_p = jnp.zeros((_EPAD, _DEDGE), jnp.float32).at[:_E].set(eas).reshape(-1)
    offs = jnp.searchsorted(dsts, jnp.arange(_NPAD + 128, dtype=jnp.int32),
                            side="left").astype(jnp.int32)

    h = jnp.zeros((_NPAD, _DIN), jnp.float32).at[:_N].set(x)

    pps = [_pad_params(p) for p in params]
    h = _layer(h, srcs_p, dsts_p, eas_p, offs, pps[0], "first")
    for i in range(1, 1 + _NMID):
        h = _layer(h, srcs_p, dsts_p, eas_p, offs, pps[i], "mid", identity=h)
    out = _layer(h, srcs_p, dsts_p, eas_p, offs, pps[-1], "last")
    return out[:_N, :3]
